# baseline (speedup 1.0000x reference)
import functools
import math

import jax
import jax.numpy as jnp
from jax import lax
from jax.experimental import pallas as pl
from jax.experimental.pallas import tpu as pltpu
from jax.experimental.pallas import tpu_sc as plsc

D_MODEL = 512
SCALE = math.sqrt(float(D_MODEL))


@functools.cache
def _make_kernel(B: int, C: int):
    info = plsc.get_sparse_core_info()
    NC, NS, L = info.num_cores, info.num_subcores, info.num_lanes
    NW = NC * NS
    assert B % NW == 0
    b_per_w = B // NW
    assert b_per_w % C == 0 and C % 8 == 0
    n_chunks = b_per_w // C
    mesh = plsc.VectorSubcoreMesh(core_axis_name="c", subcore_axis_name="s")

    @functools.partial(
        pl.kernel,
        mesh=mesh,
        out_type=jax.ShapeDtypeStruct((B, D_MODEL), jnp.float32),
        scratch_types=[
            pltpu.VMEM((C,), jnp.int32),
            pltpu.VMEM((C, D_MODEL), jnp.float32),
            pltpu.SemaphoreType.DMA,
        ],
    )
    def emb_kernel(idx_hbm, table_hbm, out_hbm, idx_v, rows_v, sem):
        wid = lax.axis_index("s") * NC + lax.axis_index("c")
        base = wid * b_per_w

        def chunk_body(g, _):
            off = base + g * C
            pltpu.sync_copy(idx_hbm.at[pl.ds(off, C)], idx_v)
            pltpu.async_copy(table_hbm.at[idx_v], rows_v, sem).wait()

            def scale_row(r, _):
                def scale_vec(d, _):
                    sl = pl.ds(d * L, L)
                    rows_v[r, sl] = rows_v[r, sl] * SCALE
                    return 0

                return lax.fori_loop(0, D_MODEL // L, scale_vec, 0)

            lax.fori_loop(0, C, scale_row, 0)
            pltpu.sync_copy(rows_v, out_hbm.at[pl.ds(off, C)])
            return 0

        lax.fori_loop(0, n_chunks, chunk_body, 0)

    return emb_kernel


def kernel(input, W):
    B = input.shape[0] * input.shape[1]
    idx = input.reshape(B).astype(jnp.int32)
    out = _make_kernel(B, 80)(idx, W)
    return out.reshape(input.shape[0], input.shape[1], D_MODEL)


# speedup vs baseline: 2.1357x; 2.1357x over previous
import functools
import math

import jax
import jax.numpy as jnp
from jax import lax
from jax.experimental import pallas as pl
from jax.experimental.pallas import tpu as pltpu
from jax.experimental.pallas import tpu_sc as plsc

D_MODEL = 512
SCALE = math.sqrt(float(D_MODEL))


@functools.cache
def _make_kernel(B: int, C: int):
    info = plsc.get_sparse_core_info()
    NC, NS, L = info.num_cores, info.num_subcores, info.num_lanes
    NW = NC * NS
    assert B % NW == 0
    b_per_w = B // NW
    assert b_per_w % C == 0 and C % 8 == 0
    n_chunks = b_per_w // C
    assert n_chunks % 2 == 0 and n_chunks >= 4
    n_slices = D_MODEL // L
    mesh = plsc.VectorSubcoreMesh(core_axis_name="c", subcore_axis_name="s")

    @functools.partial(
        pl.kernel,
        mesh=mesh,
        out_type=jax.ShapeDtypeStruct((B, D_MODEL), jnp.float32),
        scratch_types=[
            pltpu.VMEM((n_chunks, C), jnp.int32),
            pltpu.VMEM((C, D_MODEL), jnp.float32),
            pltpu.VMEM((C, D_MODEL), jnp.float32),
            pltpu.VMEM((C, D_MODEL), jnp.float32),
            pltpu.VMEM((C, D_MODEL), jnp.float32),
            pltpu.SemaphoreType.DMA,
            pltpu.SemaphoreType.DMA,
            pltpu.SemaphoreType.DMA,
            pltpu.SemaphoreType.DMA,
        ],
    )
    def emb_kernel(idx_hbm, table_hbm, out_hbm, idx_v, g0, g1, s0, s1,
                   gsem0, gsem1, ssem0, ssem1):
        wid = lax.axis_index("s") * NC + lax.axis_index("c")
        base = wid * b_per_w
        gbuf = (g0, g1)
        sbuf = (s0, s1)
        gsem = (gsem0, gsem1)
        ssem = (ssem0, ssem1)

        pltpu.sync_copy(idx_hbm.at[pl.ds(wid * n_chunks, n_chunks)], idx_v)

        def issue_gather(g, slot):
            pltpu.async_copy(table_hbm.at[idx_v.at[g]], gbuf[slot], gsem[slot])

        def wait_gather(slot):
            pltpu.make_async_copy(
                table_hbm.at[idx_v.at[0]], gbuf[slot], gsem[slot]).wait()

        def issue_scatter(g, slot):
            pltpu.async_copy(sbuf[slot], out_hbm.at[pl.ds(base + g * C, C)],
                             ssem[slot])

        def wait_scatter(slot):
            pltpu.make_async_copy(
                sbuf[slot], out_hbm.at[pl.ds(base, C)], ssem[slot]).wait()

        def scale(slot):
            src = gbuf[slot]
            dst = sbuf[slot]

            def row(r, _):
                for d in range(n_slices):
                    sl = pl.ds(d * L, L)
                    dst[r, sl] = src[r, sl] * SCALE
                return 0

            lax.fori_loop(0, C, row, 0)

        issue_gather(0, 0)
        issue_gather(1, 1)

        def chunk_step(g, slot, first):
            wait_gather(slot)
            if not first:
                wait_scatter(slot)
            scale(slot)
            issue_gather_next = g + 2

            @pl.when(issue_gather_next < n_chunks)
            def _():
                issue_gather(issue_gather_next, slot)

            issue_scatter(g, slot)

        chunk_step(0, 0, True)
        chunk_step(1, 1, True)

        def outer(i, _):
            chunk_step(2 * i, 0, False)
            chunk_step(2 * i + 1, 1, False)
            return 0

        lax.fori_loop(1, n_chunks // 2, outer, 0)

        wait_scatter(0)
        wait_scatter(1)

    return emb_kernel


def kernel(input, W):
    B = input.shape[0] * input.shape[1]
    C = 40
    idx = input.reshape(B // C, C).astype(jnp.int32)
    out = _make_kernel(B, C)(idx, W)
    return out.reshape(input.shape[0], input.shape[1], D_MODEL)


# speedup vs baseline: 3.1433x; 1.4718x over previous
import functools
import math

import jax
import jax.numpy as jnp
from jax import lax
from jax.experimental import pallas as pl
from jax.experimental.pallas import tpu as pltpu
from jax.experimental.pallas import tpu_sc as plsc

D_MODEL = 512
SCALE = math.sqrt(float(D_MODEL))


@functools.cache
def _make_kernel(S: int, T: int):
    info = plsc.get_sparse_core_info()
    NC, NS, L = info.num_cores, info.num_subcores, info.num_lanes
    NW = NC * NS
    assert S % NW == 0
    s_per_w = S // NW
    assert s_per_w % 2 == 0 and s_per_w >= 4
    n_slices = D_MODEL // L
    mesh = plsc.VectorSubcoreMesh(core_axis_name="c", subcore_axis_name="s")

    @functools.partial(
        pl.kernel,
        mesh=mesh,
        out_type=jax.ShapeDtypeStruct((S, T, D_MODEL), jnp.float32),
        scratch_types=[
            pltpu.VMEM((s_per_w, 64), jnp.int32),
            pltpu.VMEM((T, D_MODEL), jnp.float32),
            pltpu.VMEM((T, D_MODEL), jnp.float32),
            pltpu.VMEM((T, D_MODEL), jnp.float32),
            pltpu.VMEM((T, D_MODEL), jnp.float32),
            pltpu.SemaphoreType.DMA,
            pltpu.SemaphoreType.DMA,
            pltpu.SemaphoreType.DMA,
            pltpu.SemaphoreType.DMA,
        ],
    )
    def emb_kernel(idx_hbm, table_hbm, out_hbm, idx_v, g0, g1, s0, s1,
                   gsem0, gsem1, ssem0, ssem1):
        wid = lax.axis_index("s") * NC + lax.axis_index("c")
        base = wid * s_per_w
        gbuf = (g0, g1)
        sbuf = (s0, s1)
        gsem = (gsem0, gsem1)
        ssem = (ssem0, ssem1)

        pltpu.sync_copy(idx_hbm.at[pl.ds(base, s_per_w)], idx_v)

        def issue_gather(g, slot):
            pltpu.async_copy(table_hbm.at[idx_v.at[g, pl.ds(0, T)]],
                             gbuf[slot], gsem[slot])

        def wait_gather(slot):
            pltpu.make_async_copy(
                table_hbm.at[idx_v.at[0, pl.ds(0, T)]],
                gbuf[slot], gsem[slot]).wait()

        def issue_scatter(g, slot):
            pltpu.async_copy(sbuf[slot], out_hbm.at[base + g], ssem[slot])

        def wait_scatter(slot):
            pltpu.make_async_copy(
                sbuf[slot], out_hbm.at[base], ssem[slot]).wait()

        def scale(slot):
            src = gbuf[slot]
            dst = sbuf[slot]

            def row(r, _):
                for d in range(n_slices):
                    sl = pl.ds(d * L, L)
                    dst[r, sl] = src[r, sl] * SCALE
                return 0

            lax.fori_loop(0, T, row, 0)

        issue_gather(0, 0)
        issue_gather(1, 1)

        def chunk_step(g, slot, first):
            wait_gather(slot)
            if not first:
                wait_scatter(slot)
            scale(slot)
            nxt = g + 2

            @pl.when(nxt < s_per_w)
            def _():
                issue_gather(nxt, slot)

            issue_scatter(g, slot)

        chunk_step(0, 0, True)
        chunk_step(1, 1, True)

        def outer(i, _):
            chunk_step(2 * i, 0, False)
            chunk_step(2 * i + 1, 1, False)
            return 0

        lax.fori_loop(1, s_per_w // 2, outer, 0)

        wait_scatter(0)
        wait_scatter(1)

    return emb_kernel


def kernel(input, W):
    S, T = input.shape
    idx = jnp.pad(input.astype(jnp.int32), ((0, 0), (0, 64 - T)))
    return _make_kernel(S, T)(idx, W)


# speedup vs baseline: 6.5925x; 2.0973x over previous
import functools
import math

import jax
import jax.numpy as jnp
from jax import lax
from jax.experimental import pallas as pl
from jax.experimental.pallas import tpu as pltpu
from jax.experimental.pallas import tpu_sc as plsc

D_MODEL = 512
SCALE = math.sqrt(float(D_MODEL))


@functools.cache
def _make_kernel(S: int, T: int, CS: int):
    info = plsc.get_sparse_core_info()
    NC, NS, L = info.num_cores, info.num_subcores, info.num_lanes
    NW = NC * NS
    assert S % NW == 0
    s_per_w = S // NW
    assert s_per_w % CS == 0 and CS % 8 == 0
    nq = s_per_w // CS
    n_chunks = T * nq
    assert n_chunks >= 6
    n_slices = D_MODEL // L
    mesh = plsc.VectorSubcoreMesh(core_axis_name="c", subcore_axis_name="s")

    @functools.partial(
        pl.kernel,
        mesh=mesh,
        out_type=jax.ShapeDtypeStruct((T, S, D_MODEL), jnp.float32),
        scratch_types=[
            pltpu.VMEM((T, s_per_w), jnp.int32),
            pltpu.VMEM((CS, D_MODEL), jnp.float32),
            pltpu.VMEM((CS, D_MODEL), jnp.float32),
            pltpu.VMEM((CS, D_MODEL), jnp.float32),
            pltpu.SemaphoreType.DMA,
            pltpu.SemaphoreType.DMA,
            pltpu.SemaphoreType.DMA,
            pltpu.SemaphoreType.DMA,
            pltpu.SemaphoreType.DMA,
            pltpu.SemaphoreType.DMA,
        ],
    )
    def emb_kernel(idx_hbm, table_hbm, out_hbm, idx_v, b0, b1, b2,
                   gsem0, gsem1, gsem2, ssem0, ssem1, ssem2):
        wid = lax.axis_index("s") * NC + lax.axis_index("c")
        sbase = wid * s_per_w
        buf = (b0, b1, b2)
        gsem = (gsem0, gsem1, gsem2)
        ssem = (ssem0, ssem1, ssem2)

        pltpu.sync_copy(idx_hbm.at[wid], idx_v)

        def issue_gather(g, slot):
            t = g // nq
            q = g % nq
            pltpu.async_copy(
                table_hbm.at[idx_v.at[t, pl.ds(q * CS, CS)]],
                buf[slot], gsem[slot])

        def wait_gather(slot):
            pltpu.make_async_copy(
                table_hbm.at[idx_v.at[0, pl.ds(0, CS)]],
                buf[slot], gsem[slot]).wait()

        def issue_scatter(g, slot):
            t = g // nq
            q = g % nq
            pltpu.async_copy(
                buf[slot], out_hbm.at[t, pl.ds(sbase + q * CS, CS)],
                ssem[slot])

        def wait_scatter(slot):
            pltpu.make_async_copy(
                buf[slot], out_hbm.at[0, pl.ds(sbase, CS)],
                ssem[slot]).wait()

        def scale(slot):
            b = buf[slot]

            def row(r, _):
                for d in range(n_slices):
                    sl = pl.ds(d * L, L)
                    b[r, sl] = b[r, sl] * SCALE
                return 0

            lax.fori_loop(0, CS, row, 0)

        issue_gather(0, 0)
        issue_gather(1, 1)

        def chunk_step(g, slot, first):
            nslot = (slot + 2) % 3
            if not first:
                wait_scatter(nslot)
            nxt = g + 2

            @pl.when(nxt < n_chunks)
            def _():
                issue_gather(nxt, nslot)

            wait_gather(slot)
            scale(slot)
            issue_scatter(g, slot)

        chunk_step(0, 0, True)
        chunk_step(1, 1, False)
        chunk_step(2, 2, False)

        n_triples = n_chunks // 3
        rem = n_chunks - 3 * n_triples

        def outer(i, _):
            chunk_step(3 * i, 0, False)
            chunk_step(3 * i + 1, 1, False)
            chunk_step(3 * i + 2, 2, False)
            return 0

        lax.fori_loop(1, n_triples, outer, 0)

        for r in range(rem):
            g = 3 * n_triples + r
            chunk_step(g, g % 3, False)

        wait_scatter((n_chunks - 1) % 3)

    return emb_kernel


def kernel(input, W):
    S, T = input.shape
    NW = 32
    idx = (input.astype(jnp.int32).T
           .reshape(T, NW, S // NW)
           .transpose(1, 0, 2))
    out_t = _make_kernel(S, T, 64)(idx, W)
    return out_t.transpose(1, 0, 2)


# speedup vs baseline: 6.7841x; 1.0291x over previous
import functools
import math

import jax
import jax.numpy as jnp
from jax import lax
from jax.experimental import pallas as pl
from jax.experimental.pallas import tpu as pltpu
from jax.experimental.pallas import tpu_sc as plsc

D_MODEL = 512
SCALE = math.sqrt(float(D_MODEL))


@functools.cache
def _make_kernel(S: int, T: int, CS: int):
    info = plsc.get_sparse_core_info()
    NC, NS, L = info.num_cores, info.num_subcores, info.num_lanes
    NW = NC * NS
    assert S % NW == 0
    s_per_w = S // NW
    assert s_per_w % CS == 0 and CS % 8 == 0
    nq = s_per_w // CS
    n_chunks = T * nq
    assert n_chunks % 2 == 0 and n_chunks >= 4
    n_slices = D_MODEL // L
    mesh = plsc.VectorSubcoreMesh(core_axis_name="c", subcore_axis_name="s")

    @functools.partial(
        pl.kernel,
        mesh=mesh,
        out_type=jax.ShapeDtypeStruct((T, S, D_MODEL), jnp.float32),
        scratch_types=[
            pltpu.VMEM((T, s_per_w), jnp.int32),
            pltpu.VMEM((CS, D_MODEL), jnp.float32),
            pltpu.VMEM((CS, D_MODEL), jnp.float32),
            pltpu.VMEM((CS, D_MODEL), jnp.float32),
            pltpu.VMEM((CS, D_MODEL), jnp.float32),
            pltpu.VMEM((CS, D_MODEL), jnp.float32),
            pltpu.VMEM((CS, D_MODEL), jnp.float32),
            pltpu.SemaphoreType.DMA,
            pltpu.SemaphoreType.DMA,
            pltpu.SemaphoreType.DMA,
            pltpu.SemaphoreType.DMA,
            pltpu.SemaphoreType.DMA,
            pltpu.SemaphoreType.DMA,
        ],
    )
    def emb_kernel(idx_hbm, table_hbm, out_hbm, idx_v, g0, g1, g2,
                   s0, s1, s2, gsem0, gsem1, gsem2, ssem0, ssem1, ssem2):
        wid = lax.axis_index("s") * NC + lax.axis_index("c")
        sbase = wid * s_per_w
        gbuf = (g0, g1, g2)
        sbuf = (s0, s1, s2)
        gsem = (gsem0, gsem1, gsem2)
        ssem = (ssem0, ssem1, ssem2)

        pltpu.sync_copy(idx_hbm.at[wid], idx_v)

        def issue_gather(g, slot):
            t = g // nq
            q = g % nq
            pltpu.async_copy(
                table_hbm.at[idx_v.at[t, pl.ds(q * CS, CS)]],
                gbuf[slot], gsem[slot])

        def wait_gather(slot):
            pltpu.make_async_copy(
                table_hbm.at[idx_v.at[0, pl.ds(0, CS)]],
                gbuf[slot], gsem[slot]).wait()

        def issue_scatter(g, slot):
            t = g // nq
            q = g % nq
            pltpu.async_copy(
                sbuf[slot], out_hbm.at[t, pl.ds(sbase + q * CS, CS)],
                ssem[slot])

        def wait_scatter(slot):
            pltpu.make_async_copy(
                sbuf[slot], out_hbm.at[0, pl.ds(sbase, CS)],
                ssem[slot]).wait()

        def scale(slot):
            src = gbuf[slot]
            dst = sbuf[slot]

            def row(r, _):
                for d in range(n_slices):
                    sl = pl.ds(d * L, L)
                    dst[r, sl] = src[r, sl] * SCALE
                return 0

            lax.fori_loop(0, CS, row, 0)

        issue_gather(0, 0)
        issue_gather(1, 1)
        issue_gather(2, 2)

        def chunk_step(g, slot, first):
            wait_gather(slot)
            if not first:
                wait_scatter(slot)
            scale(slot)
            nxt = g + 3

            @pl.when(nxt < n_chunks)
            def _():
                issue_gather(nxt, slot)

            issue_scatter(g, slot)

        chunk_step(0, 0, True)
        chunk_step(1, 1, True)
        chunk_step(2, 2, True)

        n_triples = n_chunks // 3
        rem = n_chunks - 3 * n_triples

        def outer(i, _):
            chunk_step(3 * i, 0, False)
            chunk_step(3 * i + 1, 1, False)
            chunk_step(3 * i + 2, 2, False)
            return 0

        lax.fori_loop(1, n_triples, outer, 0)

        for r in range(rem):
            g = 3 * n_triples + r
            chunk_step(g, g % 3, False)

        wait_scatter(0)
        wait_scatter(1)
        wait_scatter(2)

    return emb_kernel


def kernel(input, W):
    S, T = input.shape
    NW = 32
    idx = (input.astype(jnp.int32).T
           .reshape(T, NW, S // NW)
           .transpose(1, 0, 2))
    out_t = _make_kernel(S, T, 32)(idx, W)
    return out_t.transpose(1, 0, 2)
